# Initial kernel scaffold; baseline (speedup 1.0000x reference)
import jax
import jax.numpy as jnp
from jax import lax
from jax.experimental import pallas as pl
from jax.experimental.pallas import tpu as pltpu
from jax.experimental.pallas import tpu_sc as plsc

N = 10000
E = 320000
D = 128

NC = 2
NS = 16
CB = 128
CH = 80
EPT = CH * CB
EPAD = NC * NS * EPT
NPAD = 10240
RPT = NPAD // NS

BM = 2000

_mesh = plsc.VectorSubcoreMesh(
    core_axis_name="c", subcore_axis_name="s", num_cores=NC, num_subcores=NS
)


def _deg_body(dst_hbm, ones_hbm, zer_hbm, out_hbm, dstv, onesv, acc):
    cid = lax.axis_index("c")
    sid = lax.axis_index("s")
    pltpu.sync_copy(zer_hbm, acc.at[pl.ds(sid * RPT, RPT)])
    pltpu.sync_copy(dst_hbm.at[cid, sid], dstv)
    pltpu.sync_copy(ones_hbm, onesv)
    plsc.subcore_barrier()

    def step(j, carry):
        pltpu.sync_copy(onesv, acc.at[dstv.at[j]], add=True)
        return carry

    lax.fori_loop(0, CH, step, 0)
    plsc.subcore_barrier()
    pltpu.sync_copy(
        acc.at[pl.ds(sid * RPT, RPT)], out_hbm.at[cid, pl.ds(sid * RPT, RPT)]
    )


_deg_call = pl.kernel(
    _deg_body,
    out_type=jax.ShapeDtypeStruct((NC, NPAD), jnp.float32),
    mesh=_mesh,
    scratch_types=[
        pltpu.VMEM((CH, CB), jnp.int32),
        pltpu.VMEM((CB,), jnp.float32),
        pltpu.VMEM_SHARED((NPAD,), jnp.float32),
    ],
)


def _agg_body(g_hbm, src_hbm, dst_hbm, zer_hbm, out_hbm,
              srcv, dstv, buf0, buf1, acc, sem0, sem1):
    cid = lax.axis_index("c")
    sid = lax.axis_index("s")
    pltpu.sync_copy(zer_hbm, acc.at[pl.ds(sid * RPT, RPT)])
    pltpu.sync_copy(src_hbm.at[cid, sid], srcv)
    pltpu.sync_copy(dst_hbm.at[cid, sid], dstv)
    plsc.subcore_barrier()

    pltpu.async_copy(g_hbm.at[srcv.at[0]], buf0, sem0)
    pltpu.async_copy(g_hbm.at[srcv.at[1]], buf1, sem1)

    def step(i, carry):
        ja = 2 * i
        jb = ja + 1
        pltpu.make_async_copy(g_hbm.at[srcv.at[ja]], buf0, sem0).wait()
        pltpu.sync_copy(buf0, acc.at[dstv.at[ja]], add=True)
        na = jnp.minimum(ja + 2, CH - 1)
        pltpu.async_copy(g_hbm.at[srcv.at[na]], buf0, sem0)
        pltpu.make_async_copy(g_hbm.at[srcv.at[jb]], buf1, sem1).wait()
        pltpu.sync_copy(buf1, acc.at[dstv.at[jb]], add=True)
        nb = jnp.minimum(jb + 2, CH - 1)
        pltpu.async_copy(g_hbm.at[srcv.at[nb]], buf1, sem1)
        return carry

    lax.fori_loop(0, CH // 2, step, 0)
    pltpu.make_async_copy(g_hbm.at[srcv.at[CH - 1]], buf0, sem0).wait()
    pltpu.make_async_copy(g_hbm.at[srcv.at[CH - 1]], buf1, sem1).wait()
    plsc.subcore_barrier()
    pltpu.sync_copy(
        acc.at[pl.ds(sid * RPT, RPT)], out_hbm.at[cid, pl.ds(sid * RPT, RPT)]
    )


_agg_call = pl.kernel(
    _agg_body,
    out_type=jax.ShapeDtypeStruct((NC, NPAD, D), jnp.float32),
    mesh=_mesh,
    scratch_types=[
        pltpu.VMEM((CH, CB), jnp.int32),
        pltpu.VMEM((CH, CB), jnp.int32),
        pltpu.VMEM((CB, D), jnp.float32),
        pltpu.VMEM((CB, D), jnp.float32),
        pltpu.VMEM_SHARED((NPAD, D), jnp.float32),
        pltpu.SemaphoreType.DMA,
        pltpu.SemaphoreType.DMA,
    ],
)


def _tc_first_body(x_ref, w_ref, ds_ref, g_ref):
    dinv = lax.rsqrt(ds_ref[...])
    h = jnp.dot(x_ref[...], w_ref[...], preferred_element_type=jnp.float32)
    g_ref[...] = h * dinv


def _tc_mid_body(a0_ref, a1_ref, g_ref, ds_ref, b_ref, w_ref, o_ref):
    dinv = lax.rsqrt(ds_ref[...])
    z = (a0_ref[...] + a1_ref[...] + g_ref[...]) * dinv + b_ref[...]
    z = jnp.maximum(z, 0.0)
    h = jnp.dot(z, w_ref[...], preferred_element_type=jnp.float32)
    o_ref[...] = h * dinv


def _tc_last_body(a0_ref, a1_ref, g_ref, ds_ref, b_ref, o_ref):
    dinv = lax.rsqrt(ds_ref[...])
    o_ref[...] = (a0_ref[...] + a1_ref[...] + g_ref[...]) * dinv + b_ref[...]


_row_spec = pl.BlockSpec((BM, D), lambda i: (i, 0))
_col_spec = pl.BlockSpec((BM, 1), lambda i: (i, 0))
_w_spec = pl.BlockSpec((D, D), lambda i: (0, 0))
_b_spec = pl.BlockSpec((1, D), lambda i: (0, 0))
_out_sd = jax.ShapeDtypeStruct((N, D), jnp.float32)

_tc_first = pl.pallas_call(
    _tc_first_body,
    grid=(N // BM,),
    in_specs=[_row_spec, _w_spec, _col_spec],
    out_specs=_row_spec,
    out_shape=_out_sd,
)

_tc_mid = pl.pallas_call(
    _tc_mid_body,
    grid=(N // BM,),
    in_specs=[_row_spec, _row_spec, _row_spec, _col_spec, _b_spec, _w_spec],
    out_specs=_row_spec,
    out_shape=_out_sd,
)

_tc_last = pl.pallas_call(
    _tc_last_body,
    grid=(N // BM,),
    in_specs=[_row_spec, _row_spec, _row_spec, _col_spec, _b_spec],
    out_specs=_row_spec,
    out_shape=_out_sd,
)


def kernel(x, edge_index, W1, b1, W2, b2):
    src = edge_index[0].astype(jnp.int32)
    dst = edge_index[1].astype(jnp.int32)
    npad = EPAD - E
    pad_ar = jnp.arange(npad, dtype=jnp.int32)
    pad_src = (pad_ar * 37) % N
    pad_dst = N + pad_ar % (NPAD - N)
    srcp = jnp.concatenate([src, pad_src]).reshape(NC, NS, CH, CB)
    dstp = jnp.concatenate([dst, pad_dst]).reshape(NC, NS, CH, CB)

    ones_v = jnp.ones((CB,), jnp.float32)
    zer_1 = jnp.zeros((RPT,), jnp.float32)
    zer_2 = jnp.zeros((RPT, D), jnp.float32)

    degp = _deg_call(dstp, ones_v, zer_1)
    dsum = (degp[0, :N] + degp[1, :N] + 1.0)[:, None]

    b1r = b1.reshape(1, D)
    b2r = b2.reshape(1, D)

    g1 = _tc_first(x, W1, dsum)
    acc1 = _agg_call(g1, srcp, dstp, zer_2)
    g2 = _tc_mid(acc1[0, :N], acc1[1, :N], g1, dsum, b1r, W2)
    acc2 = _agg_call(g2, srcp, dstp, zer_2)
    out = _tc_last(acc2[0, :N], acc2[1, :N], g2, dsum, b2r)
    return out


# speedup vs baseline: 25.0242x; 25.0242x over previous
import jax
import jax.numpy as jnp
from jax import lax
from jax.experimental import pallas as pl
from jax.experimental.pallas import tpu as pltpu
from jax.experimental.pallas import tpu_sc as plsc

N = 10000
E = 320000
D = 128
DH = D // 2

NC = 2
NS = 16
CB = 128
CH = 160
DCH = 80
EPT = CH * CB
EPAD = NS * EPT
NPAD = 10240
RPT = NPAD // NS

BM = 2000

_mesh = plsc.VectorSubcoreMesh(
    core_axis_name="c", subcore_axis_name="s", num_cores=NC, num_subcores=NS
)

_sc_params = pltpu.CompilerParams(use_tc_tiling_on_sc=False)


def _deg_body(dst_hbm, ones_hbm, zer_hbm, out_hbm, dstv, onesv, acc):
    cid = lax.axis_index("c")
    sid = lax.axis_index("s")
    pltpu.sync_copy(zer_hbm, acc.at[pl.ds(sid * RPT, RPT)])
    pltpu.sync_copy(dst_hbm.at[cid, sid], dstv)
    pltpu.sync_copy(ones_hbm, onesv)
    plsc.subcore_barrier()

    def step(j, carry):
        pltpu.sync_copy(onesv, acc.at[dstv.at[j]], add=True)
        return carry

    lax.fori_loop(0, DCH, step, 0)
    plsc.subcore_barrier()
    pltpu.sync_copy(
        acc.at[pl.ds(sid * RPT, RPT)], out_hbm.at[cid, pl.ds(sid * RPT, RPT)]
    )


_deg_call = pl.kernel(
    _deg_body,
    out_type=jax.ShapeDtypeStruct((NC, NPAD), jnp.float32),
    mesh=_mesh,
    scratch_types=[
        pltpu.VMEM((DCH, CB), jnp.int32),
        pltpu.VMEM((CB,), jnp.float32),
        pltpu.VMEM_SHARED((NPAD,), jnp.float32),
    ],
    compiler_params=_sc_params,
)


def _agg_body(g_hbm, src_hbm, dst_hbm, zer_hbm, out_hbm,
              srcv, dstv, buf0, buf1, acc, sem0, sem1):
    cid = lax.axis_index("c")
    sid = lax.axis_index("s")
    pltpu.sync_copy(zer_hbm, acc.at[pl.ds(sid * RPT, RPT)])
    pltpu.sync_copy(src_hbm.at[sid], srcv)
    pltpu.sync_copy(dst_hbm.at[sid], dstv)
    plsc.subcore_barrier()

    gc = g_hbm.at[cid]

    pltpu.async_copy(gc.at[srcv.at[0]], buf0, sem0)
    pltpu.async_copy(gc.at[srcv.at[1]], buf1, sem1)

    def step(i, carry):
        ja = 2 * i
        jb = ja + 1
        pltpu.make_async_copy(gc.at[srcv.at[ja]], buf0, sem0).wait()
        pltpu.sync_copy(buf0, acc.at[dstv.at[ja]], add=True)
        na = jnp.minimum(ja + 2, CH - 1)
        pltpu.async_copy(gc.at[srcv.at[na]], buf0, sem0)
        pltpu.make_async_copy(gc.at[srcv.at[jb]], buf1, sem1).wait()
        pltpu.sync_copy(buf1, acc.at[dstv.at[jb]], add=True)
        nb = jnp.minimum(jb + 2, CH - 1)
        pltpu.async_copy(gc.at[srcv.at[nb]], buf1, sem1)
        return carry

    lax.fori_loop(0, CH // 2, step, 0)
    pltpu.make_async_copy(gc.at[srcv.at[CH - 1]], buf0, sem0).wait()
    pltpu.make_async_copy(gc.at[srcv.at[CH - 1]], buf1, sem1).wait()
    plsc.subcore_barrier()
    pltpu.sync_copy(
        acc.at[pl.ds(sid * RPT, RPT)], out_hbm.at[cid, pl.ds(sid * RPT, RPT)]
    )


_agg_call = pl.kernel(
    _agg_body,
    out_type=jax.ShapeDtypeStruct((NC, NPAD, DH), jnp.float32),
    mesh=_mesh,
    scratch_types=[
        pltpu.VMEM((CH, CB), jnp.int32),
        pltpu.VMEM((CH, CB), jnp.int32),
        pltpu.VMEM((CB, DH), jnp.float32),
        pltpu.VMEM((CB, DH), jnp.float32),
        pltpu.VMEM_SHARED((NPAD, DH), jnp.float32),
        pltpu.SemaphoreType.DMA,
        pltpu.SemaphoreType.DMA,
    ],
    compiler_params=_sc_params,
)


def _tc_first_body(x_ref, w_ref, ds_ref, g_ref):
    dinv = lax.rsqrt(ds_ref[...])
    h = jnp.dot(x_ref[...], w_ref[...], preferred_element_type=jnp.float32)
    g_ref[...] = h * dinv


def _tc_mid_body(a_ref, g_ref, ds_ref, b_ref, w_ref, o_ref):
    dinv = lax.rsqrt(ds_ref[...])
    z = (a_ref[...] + g_ref[...]) * dinv + b_ref[...]
    z = jnp.maximum(z, 0.0)
    h = jnp.dot(z, w_ref[...], preferred_element_type=jnp.float32)
    o_ref[...] = h * dinv


def _tc_last_body(a_ref, g_ref, ds_ref, b_ref, o_ref):
    dinv = lax.rsqrt(ds_ref[...])
    o_ref[...] = (a_ref[...] + g_ref[...]) * dinv + b_ref[...]


_row_spec = pl.BlockSpec((BM, D), lambda i: (i, 0))
_col_spec = pl.BlockSpec((BM, 1), lambda i: (i, 0))
_w_spec = pl.BlockSpec((D, D), lambda i: (0, 0))
_b_spec = pl.BlockSpec((1, D), lambda i: (0, 0))
_out_sd = jax.ShapeDtypeStruct((N, D), jnp.float32)

_tc_first = pl.pallas_call(
    _tc_first_body,
    grid=(N // BM,),
    in_specs=[_row_spec, _w_spec, _col_spec],
    out_specs=_row_spec,
    out_shape=_out_sd,
)

_tc_mid = pl.pallas_call(
    _tc_mid_body,
    grid=(N // BM,),
    in_specs=[_row_spec, _row_spec, _col_spec, _b_spec, _w_spec],
    out_specs=_row_spec,
    out_shape=_out_sd,
)

_tc_last = pl.pallas_call(
    _tc_last_body,
    grid=(N // BM,),
    in_specs=[_row_spec, _row_spec, _col_spec, _b_spec],
    out_specs=_row_spec,
    out_shape=_out_sd,
)


def _split_halves(g):
    return jnp.stack([g[:, :DH], g[:, DH:]])


def _join_halves(a):
    return jnp.concatenate([a[0, :N], a[1, :N]], axis=1)


def kernel(x, edge_index, W1, b1, W2, b2):
    src = edge_index[0].astype(jnp.int32)
    dst = edge_index[1].astype(jnp.int32)
    npad = EPAD - E
    pad_ar = jnp.arange(npad, dtype=jnp.int32)
    pad_src = (pad_ar * 37) % N
    pad_dst = N + pad_ar % (NPAD - N)
    srca = jnp.concatenate([src, pad_src]).reshape(NS, CH, CB)
    dsta = jnp.concatenate([dst, pad_dst]).reshape(NS, CH, CB)
    dstd = dsta.reshape(NC, NS, DCH, CB)

    ones_v = jnp.ones((CB,), jnp.float32)
    zer_1 = jnp.zeros((RPT,), jnp.float32)
    zer_2 = jnp.zeros((RPT, DH), jnp.float32)

    degp = _deg_call(dstd, ones_v, zer_1)
    dsum = (degp[0, :N] + degp[1, :N] + 1.0)[:, None]

    b1r = b1.reshape(1, D)
    b2r = b2.reshape(1, D)

    g1 = _tc_first(x, W1, dsum)
    acc1 = _agg_call(_split_halves(g1), srca, dsta, zer_2)
    g2 = _tc_mid(_join_halves(acc1), g1, dsum, b1r, W2)
    acc2 = _agg_call(_split_halves(g2), srca, dsta, zer_2)
    out = _tc_last(_join_halves(acc2), g2, dsum, b2r)
    return out


# speedup vs baseline: 25.9149x; 1.0356x over previous
import jax
import jax.numpy as jnp
from jax import lax
from jax.experimental import pallas as pl
from jax.experimental.pallas import tpu as pltpu
from jax.experimental.pallas import tpu_sc as plsc

N = 10000
E = 320000
D = 128
DH = D // 2

NC = 2
NS = 16
CB = 128
CH = 160
DCH = 80
EPT = CH * CB
EPAD = NS * EPT
NPAD = 10240
RPT = NPAD // NS

BM = 2000

_mesh = plsc.VectorSubcoreMesh(
    core_axis_name="c", subcore_axis_name="s", num_cores=NC, num_subcores=NS
)

_sc_params = pltpu.CompilerParams(use_tc_tiling_on_sc=False)


def _deg_body(dst_hbm, ones_hbm, zer_hbm, out_hbm, dstv, onesv, acc):
    cid = lax.axis_index("c")
    sid = lax.axis_index("s")
    pltpu.sync_copy(zer_hbm, acc.at[pl.ds(sid * RPT, RPT)])
    pltpu.sync_copy(dst_hbm.at[cid, sid], dstv)
    pltpu.sync_copy(ones_hbm, onesv)
    plsc.subcore_barrier()

    def step(j, carry):
        pltpu.sync_copy(onesv, acc.at[dstv.at[j]], add=True)
        return carry

    lax.fori_loop(0, DCH, step, 0)
    plsc.subcore_barrier()
    pltpu.sync_copy(
        acc.at[pl.ds(sid * RPT, RPT)], out_hbm.at[cid, pl.ds(sid * RPT, RPT)]
    )


_deg_call = pl.kernel(
    _deg_body,
    out_type=jax.ShapeDtypeStruct((NC, NPAD), jnp.float32),
    mesh=_mesh,
    scratch_types=[
        pltpu.VMEM((DCH, CB), jnp.int32),
        pltpu.VMEM((CB,), jnp.float32),
        pltpu.VMEM_SHARED((NPAD,), jnp.float32),
    ],
    compiler_params=_sc_params,
)


def _agg_body(g_hbm, src_hbm, dst_hbm, zer_hbm, out_hbm,
              srcv, dstv, bufs, gsems, ssems, acc):
    cid = lax.axis_index("c")
    sid = lax.axis_index("s")
    pltpu.sync_copy(zer_hbm, acc.at[pl.ds(sid * RPT, RPT)])
    pltpu.sync_copy(src_hbm.at[sid], srcv)
    pltpu.sync_copy(dst_hbm.at[sid], dstv)
    plsc.subcore_barrier()

    gc = g_hbm.at[cid]

    def g_start(j, b):
        pltpu.async_copy(gc.at[srcv.at[j]], bufs[b], gsems[b])

    def g_wait(j, b):
        pltpu.make_async_copy(gc.at[srcv.at[j]], bufs[b], gsems[b]).wait()

    def s_start(j, b):
        pltpu.async_copy(bufs[b], acc.at[dstv.at[j]], ssems[b], add=True)

    def s_wait(j, b):
        pltpu.make_async_copy(bufs[b], acc.at[dstv.at[j]], ssems[b]).wait()

    g_start(0, 0)
    g_start(1, 1)

    def step(i, carry):
        for b in range(4):
            j = 4 * i + b
            g_wait(j, b)
            s_start(j, b)
            jm2 = jnp.maximum(j - 2, 0)
            if b < 2:

                @pl.when(i > 0)
                def _():
                    s_wait(jm2, (b + 2) % 4)
            else:
                s_wait(jm2, (b + 2) % 4)
            jn = jnp.minimum(j + 2, CH - 1)
            g_start(jn, (b + 2) % 4)
        return carry

    lax.fori_loop(0, CH // 4, step, 0)
    s_wait(CH - 2, (CH - 2) % 4)
    s_wait(CH - 1, (CH - 1) % 4)
    g_wait(CH - 1, CH % 4)
    g_wait(CH - 1, (CH + 1) % 4)
    plsc.subcore_barrier()
    pltpu.sync_copy(
        acc.at[pl.ds(sid * RPT, RPT)], out_hbm.at[cid, pl.ds(sid * RPT, RPT)]
    )


_agg_call = pl.kernel(
    _agg_body,
    out_type=jax.ShapeDtypeStruct((NC, NPAD, DH), jnp.float32),
    mesh=_mesh,
    scratch_types=[
        pltpu.VMEM((CH, CB), jnp.int32),
        pltpu.VMEM((CH, CB), jnp.int32),
        [pltpu.VMEM((CB, DH), jnp.float32)] * 4,
        [pltpu.SemaphoreType.DMA] * 4,
        [pltpu.SemaphoreType.DMA] * 4,
        pltpu.VMEM_SHARED((NPAD, DH), jnp.float32),
    ],
    compiler_params=_sc_params,
)


def _tc_first_body(x_ref, w_ref, ds_ref, g_ref):
    dinv = lax.rsqrt(ds_ref[...])
    h = jnp.dot(x_ref[...], w_ref[...], preferred_element_type=jnp.float32)
    g_ref[...] = h * dinv


def _tc_mid_body(a_ref, g_ref, ds_ref, b_ref, w_ref, o_ref):
    dinv = lax.rsqrt(ds_ref[...])
    z = (a_ref[...] + g_ref[...]) * dinv + b_ref[...]
    z = jnp.maximum(z, 0.0)
    h = jnp.dot(z, w_ref[...], preferred_element_type=jnp.float32)
    o_ref[...] = h * dinv


def _tc_last_body(a_ref, g_ref, ds_ref, b_ref, o_ref):
    dinv = lax.rsqrt(ds_ref[...])
    o_ref[...] = (a_ref[...] + g_ref[...]) * dinv + b_ref[...]


_row_spec = pl.BlockSpec((BM, D), lambda i: (i, 0))
_col_spec = pl.BlockSpec((BM, 1), lambda i: (i, 0))
_w_spec = pl.BlockSpec((D, D), lambda i: (0, 0))
_b_spec = pl.BlockSpec((1, D), lambda i: (0, 0))
_out_sd = jax.ShapeDtypeStruct((N, D), jnp.float32)

_tc_first = pl.pallas_call(
    _tc_first_body,
    grid=(N // BM,),
    in_specs=[_row_spec, _w_spec, _col_spec],
    out_specs=_row_spec,
    out_shape=_out_sd,
)

_tc_mid = pl.pallas_call(
    _tc_mid_body,
    grid=(N // BM,),
    in_specs=[_row_spec, _row_spec, _col_spec, _b_spec, _w_spec],
    out_specs=_row_spec,
    out_shape=_out_sd,
)

_tc_last = pl.pallas_call(
    _tc_last_body,
    grid=(N // BM,),
    in_specs=[_row_spec, _row_spec, _col_spec, _b_spec],
    out_specs=_row_spec,
    out_shape=_out_sd,
)


def _split_halves(g):
    return jnp.stack([g[:, :DH], g[:, DH:]])


def _join_halves(a):
    return jnp.concatenate([a[0, :N], a[1, :N]], axis=1)


def kernel(x, edge_index, W1, b1, W2, b2):
    src = edge_index[0].astype(jnp.int32)
    dst = edge_index[1].astype(jnp.int32)
    npad = EPAD - E
    pad_ar = jnp.arange(npad, dtype=jnp.int32)
    pad_src = (pad_ar * 37) % N
    pad_dst = N + pad_ar % (NPAD - N)
    srca = jnp.concatenate([src, pad_src]).reshape(NS, CH, CB)
    dsta = jnp.concatenate([dst, pad_dst]).reshape(NS, CH, CB)
    dstd = dsta.reshape(NC, NS, DCH, CB)

    ones_v = jnp.ones((CB,), jnp.float32)
    zer_1 = jnp.zeros((RPT,), jnp.float32)
    zer_2 = jnp.zeros((RPT, DH), jnp.float32)

    degp = _deg_call(dstd, ones_v, zer_1)
    dsum = (degp[0, :N] + degp[1, :N] + 1.0)[:, None]

    b1r = b1.reshape(1, D)
    b2r = b2.reshape(1, D)

    g1 = _tc_first(x, W1, dsum)
    acc1 = _agg_call(_split_halves(g1), srca, dsta, zer_2)
    g2 = _tc_mid(_join_halves(acc1), g1, dsum, b1r, W2)
    acc2 = _agg_call(_split_halves(g2), srca, dsta, zer_2)
    out = _tc_last(_join_halves(acc2), g2, dsum, b2r)
    return out
